# baseline (device time: 14029 ns/iter reference)
import jax
import jax.numpy as jnp
from jax import lax
from jax.experimental import pallas as pl
from jax.experimental.pallas import tpu as pltpu

S_PRE = 0.95
S_POST = 1.35


def kernel(x, dy):
    k, d = x.shape
    _, f = dy.shape
    half = d // 2
    fq = f // 4
    fh = fq // 2
    n_chunks = 2

    r_out = 2 * lax.axis_index("y") + lax.axis_index("z")
    dy_q = lax.dynamic_slice_in_dim(dy, r_out * fq, fq, axis=1).astype(
        jnp.bfloat16
    )
    x_b = x.astype(jnp.bfloat16)

    def body(x_ref, dyq_ref, out_ref, pmine_ref, sx_ref, rx_ref,
             ag_ref, ssx, rsx, sag, rag):
        my_x = lax.axis_index("x")
        my_y = lax.axis_index("y")
        my_z = lax.axis_index("z")
        r = 2 * my_y + my_z
        xp = (1 - my_x, my_y, my_z)
        zp = (my_x, my_y, 1 - my_z)
        yp = (my_x, 1 - my_y, my_z)
        dg = (my_x, 1 - my_y, 1 - my_z)

        barrier = pltpu.get_barrier_semaphore()
        for nbr in (xp, zp, yp, dg):
            pl.semaphore_signal(
                barrier, inc=1, device_id=nbr,
                device_id_type=pl.DeviceIdType.MESH,
            )

        rdma_x = []
        for c in range(n_chunks):
            cs = pl.ds(c * fh, fh)
            pq = lax.dot_general(
                x_ref[...], dyq_ref[:, cs],
                (((0,), (0,)), ((), ())),
                preferred_element_type=jnp.float32,
            )

            def quant_pre(v):
                return jnp.clip(
                    jnp.rint(v * (1.0 / S_PRE)), -127.0, 127.0
                ).astype(jnp.int8)

            @pl.when(my_x == 0)
            def _():
                pmine_ref[:, cs] = pq[:half].astype(jnp.bfloat16)
                sx_ref[c] = quant_pre(pq[half:])

            @pl.when(my_x == 1)
            def _():
                pmine_ref[:, cs] = pq[half:].astype(jnp.bfloat16)
                sx_ref[c] = quant_pre(pq[:half])

            if c == 0:
                pl.semaphore_wait(barrier, 4)
            rdma = pltpu.make_async_remote_copy(
                src_ref=sx_ref.at[c], dst_ref=rx_ref.at[c],
                send_sem=ssx.at[c], recv_sem=rsx.at[c],
                device_id=xp, device_id_type=pl.DeviceIdType.MESH,
            )
            rdma.start()
            rdma_x.append(rdma)

        ag_rdmas = []
        for c in range(n_chunks):
            cs = pl.ds(c * fh, fh)
            rdma_x[c].wait()
            u = (
                pmine_ref[:, cs].astype(jnp.float32)
                + rx_ref[c].astype(jnp.float32) * S_PRE
            )
            ag_ref[pl.ds(r, 1), :, cs] = jnp.clip(
                jnp.rint(u * (1.0 / S_POST)), -127.0, 127.0
            ).astype(jnp.int8)[None]
            for i, nbr in enumerate((zp, yp, dg)):
                s = 3 * c + i
                rdma = pltpu.make_async_remote_copy(
                    src_ref=ag_ref.at[r, :, cs],
                    dst_ref=ag_ref.at[r, :, cs],
                    send_sem=sag.at[s], recv_sem=rag.at[s],
                    device_id=nbr, device_id_type=pl.DeviceIdType.MESH,
                )
                rdma.start()
                ag_rdmas.append(rdma)
            out_ref[:, pl.ds(r * fq + c * fh, fh)] = u.astype(jnp.bfloat16)

        for c in range(n_chunks):
            cs = pl.ds(c * fh, fh)
            for i, slot in enumerate((r ^ 1, r ^ 2, r ^ 3)):
                s = 3 * c + i
                ag_rdmas[s].wait()
                out_ref[:, pl.ds(slot * fq + c * fh, fh)] = (
                    ag_ref[pl.ds(slot, 1), :, cs][0].astype(jnp.float32)
                    * S_POST
                ).astype(jnp.bfloat16)

    return pl.pallas_call(
        body,
        out_shape=jax.ShapeDtypeStruct((half, f), jnp.bfloat16),
        in_specs=[
            pl.BlockSpec(memory_space=pltpu.VMEM),
            pl.BlockSpec(memory_space=pltpu.VMEM),
        ],
        out_specs=pl.BlockSpec(memory_space=pltpu.VMEM),
        scratch_shapes=[
            pltpu.VMEM((half, fq), jnp.bfloat16),
            pltpu.VMEM((n_chunks, half, fh), jnp.int8),
            pltpu.VMEM((n_chunks, half, fh), jnp.int8),
            pltpu.VMEM((4, half, fq), jnp.int8),
            pltpu.SemaphoreType.DMA((n_chunks,)),
            pltpu.SemaphoreType.DMA((n_chunks,)),
            pltpu.SemaphoreType.DMA((6,)),
            pltpu.SemaphoreType.DMA((6,)),
        ],
        compiler_params=pltpu.CompilerParams(collective_id=0),
    )(x_b, dy_q)


# device time: 13596 ns/iter; 1.0318x vs baseline; 1.0318x over previous
import jax
import jax.numpy as jnp
from jax import lax
from jax.experimental import pallas as pl
from jax.experimental.pallas import tpu as pltpu

S_PRE = 0.95
S_POST = 1.35


def kernel(x, dy):
    k, d = x.shape
    _, f = dy.shape
    half = d // 2
    fq = f // 4
    n_chunks = 4
    fh = fq // n_chunks

    r_out = 2 * lax.axis_index("y") + lax.axis_index("z")
    dy_q = lax.dynamic_slice_in_dim(dy, r_out * fq, fq, axis=1).astype(
        jnp.bfloat16
    )
    x_b = x.astype(jnp.bfloat16)

    def body(x_ref, dyq_ref, out_ref, pmine_ref, sx_ref, rx_ref,
             ag_ref, ssx, rsx, sag, rag):
        my_x = lax.axis_index("x")
        my_y = lax.axis_index("y")
        my_z = lax.axis_index("z")
        r = 2 * my_y + my_z
        xp = (1 - my_x, my_y, my_z)
        zp = (my_x, my_y, 1 - my_z)
        yp = (my_x, 1 - my_y, my_z)
        dg = (my_x, 1 - my_y, 1 - my_z)

        barrier = pltpu.get_barrier_semaphore()
        for nbr in (xp, zp, yp, dg):
            pl.semaphore_signal(
                barrier, inc=1, device_id=nbr,
                device_id_type=pl.DeviceIdType.MESH,
            )

        rdma_x = []
        for c in range(n_chunks):
            cs = pl.ds(c * fh, fh)
            pq = lax.dot_general(
                x_ref[...], dyq_ref[:, cs],
                (((0,), (0,)), ((), ())),
                preferred_element_type=jnp.float32,
            )

            def quant_pre(v):
                return jnp.clip(
                    jnp.rint(v * (1.0 / S_PRE)), -127.0, 127.0
                ).astype(jnp.int8)

            @pl.when(my_x == 0)
            def _():
                pmine_ref[:, cs] = pq[:half].astype(jnp.bfloat16)
                sx_ref[c] = quant_pre(pq[half:])

            @pl.when(my_x == 1)
            def _():
                pmine_ref[:, cs] = pq[half:].astype(jnp.bfloat16)
                sx_ref[c] = quant_pre(pq[:half])

            if c == 0:
                pl.semaphore_wait(barrier, 4)
            rdma = pltpu.make_async_remote_copy(
                src_ref=sx_ref.at[c], dst_ref=rx_ref.at[c],
                send_sem=ssx.at[c], recv_sem=rsx.at[c],
                device_id=xp, device_id_type=pl.DeviceIdType.MESH,
            )
            rdma.start()
            rdma_x.append(rdma)

        ag_rdmas = []
        for c in range(n_chunks):
            cs = pl.ds(c * fh, fh)
            rdma_x[c].wait()
            u = (
                pmine_ref[:, cs].astype(jnp.float32)
                + rx_ref[c].astype(jnp.float32) * S_PRE
            )
            ag_ref[pl.ds(r, 1), :, cs] = jnp.clip(
                jnp.rint(u * (1.0 / S_POST)), -127.0, 127.0
            ).astype(jnp.int8)[None]
            for i, nbr in enumerate((dg, zp, yp)):
                s = 3 * c + i
                rdma = pltpu.make_async_remote_copy(
                    src_ref=ag_ref.at[r, :, cs],
                    dst_ref=ag_ref.at[r, :, cs],
                    send_sem=sag.at[s], recv_sem=rag.at[s],
                    device_id=nbr, device_id_type=pl.DeviceIdType.MESH,
                )
                rdma.start()
                ag_rdmas.append(rdma)
            out_ref[:, pl.ds(r * fq + c * fh, fh)] = u.astype(jnp.bfloat16)

        for c in range(n_chunks):
            cs = pl.ds(c * fh, fh)
            for i, slot in ((1, r ^ 1), (2, r ^ 2), (0, r ^ 3)):
                s = 3 * c + i
                ag_rdmas[s].wait()
                out_ref[:, pl.ds(slot * fq + c * fh, fh)] = (
                    ag_ref[pl.ds(slot, 1), :, cs][0].astype(jnp.float32)
                    * S_POST
                ).astype(jnp.bfloat16)

    return pl.pallas_call(
        body,
        out_shape=jax.ShapeDtypeStruct((half, f), jnp.bfloat16),
        in_specs=[
            pl.BlockSpec(memory_space=pltpu.VMEM),
            pl.BlockSpec(memory_space=pltpu.VMEM),
        ],
        out_specs=pl.BlockSpec(memory_space=pltpu.VMEM),
        scratch_shapes=[
            pltpu.VMEM((half, fq), jnp.bfloat16),
            pltpu.VMEM((n_chunks, half, fh), jnp.int8),
            pltpu.VMEM((n_chunks, half, fh), jnp.int8),
            pltpu.VMEM((4, half, fq), jnp.int8),
            pltpu.SemaphoreType.DMA((n_chunks,)),
            pltpu.SemaphoreType.DMA((n_chunks,)),
            pltpu.SemaphoreType.DMA((3 * n_chunks,)),
            pltpu.SemaphoreType.DMA((3 * n_chunks,)),
        ],
        compiler_params=pltpu.CompilerParams(collective_id=0),
    )(x_b, dy_q)
